# baseline (speedup 1.0000x reference)
import functools
import math

import jax
import jax.numpy as jnp
import numpy as np
from jax import lax
from jax.experimental import pallas as pl
from jax.experimental.pallas import tpu as pltpu
from jax.experimental.pallas import tpu_sc as plsc

VOCAB = 100000
D_MODEL = 768
MAX_POS = 2048
_SCALE = math.sqrt(float(D_MODEL))
_LANES = 16
_CHUNK = 16


def _positional_encoding_np(length, depth):
    depth_h = depth / 2
    positions = np.arange(length)[:, np.newaxis]
    depths = np.arange(depth_h)[np.newaxis, :] / depth_h
    angle_rates = 1 / 10000 ** depths
    angle_rads = positions * angle_rates
    return np.concatenate(
        [np.sin(angle_rads), np.cos(angle_rads)], axis=-1
    ).astype(np.float32)


@functools.partial(jax.jit, static_argnums=(3, 4))
def _run(xf, pos, table, batch, seq_len):
    info = plsc.get_sparse_core_info()
    nc, ns = info.num_cores, info.num_subcores
    nw = nc * ns
    t_span = seq_len // nw
    b_per_w = batch * t_span
    halves = t_span // _CHUNK
    n_chunks = batch * halves
    cols16 = D_MODEL // _LANES
    n_rows = batch * seq_len

    mesh = plsc.VectorSubcoreMesh(core_axis_name="c", subcore_axis_name="s")

    @functools.partial(
        pl.kernel,
        mesh=mesh,
        out_type=jax.ShapeDtypeStruct((n_rows, D_MODEL), jnp.float32),
        scratch_types=[
            pltpu.VMEM((b_per_w,), jnp.int32),
            pltpu.VMEM((t_span, D_MODEL), jnp.float32),
            pltpu.VMEM((_CHUNK, D_MODEL), jnp.float32),
            pltpu.VMEM((_CHUNK, D_MODEL), jnp.float32),
            pltpu.VMEM((_CHUNK, D_MODEL), jnp.float32),
            pltpu.VMEM((_CHUNK, D_MODEL), jnp.float32),
            pltpu.SemaphoreType.DMA,
            pltpu.SemaphoreType.DMA,
            pltpu.SemaphoreType.DMA,
            pltpu.SemaphoreType.DMA,
            pltpu.SemaphoreType.DMA,
        ],
    )
    def body(x_hbm, pos_hbm, table_hbm, out_hbm,
             idx_v, pos_v, g0, g1, p0, p1,
             psem, gs0, gs1, os0, os1):
        g = (g0, g1)
        p = (p0, p1)
        gsem = (gs0, gs1)
        osem = (os0, os1)
        wid = lax.axis_index("s") * nc + lax.axis_index("c")
        t0 = wid * t_span
        ph = pltpu.async_copy(pos_hbm.at[pl.ds(t0, t_span)], pos_v, psem)
        for b in range(batch):
            pltpu.sync_copy(
                x_hbm.at[pl.ds(b * seq_len + t0, t_span)],
                idx_v.at[pl.ds(b * t_span, t_span)])
        ph.wait()

        def start_gather(j):
            return pltpu.async_copy(
                table_hbm.at[idx_v.at[pl.ds(j * _CHUNK, _CHUNK)]],
                g[j % 2], gsem[j % 2])

        store_h = [None, None]
        pend = start_gather(0)
        for j in range(n_chunks):
            buf = j % 2
            b, half = j // halves, j % halves
            if j + 1 < n_chunks:
                nbuf = (j + 1) % 2
                if store_h[nbuf] is not None:
                    store_h[nbuf].wait()
                    store_h[nbuf] = None
                nxt = start_gather(j + 1)
            pend.wait()
            pr0 = half * _CHUNK

            @plsc.parallel_loop(0, _CHUNK, 1, unroll=1)
            def _(r):
                for c in range(cols16):
                    sl = pl.ds(c * _LANES, _LANES)
                    p[buf][r, sl] = g[buf][r, sl] * _SCALE + pos_v[pr0 + r, sl]

            store_h[buf] = pltpu.async_copy(
                p[buf],
                out_hbm.at[pl.ds(b * seq_len + t0 + half * _CHUNK, _CHUNK)],
                osem[buf])
            if j + 1 < n_chunks:
                pend = nxt
        for h in store_h:
            if h is not None:
                h.wait()

    return body(xf, pos, table)


def kernel(x, table):
    b, t = x.shape
    xf = x.reshape(b * t).astype(jnp.int32)
    pos = jnp.asarray(_positional_encoding_np(MAX_POS, D_MODEL))
    out = _run(xf, pos, table, b, t)
    return out.reshape(b, t, D_MODEL)


# speedup vs baseline: 1.0709x; 1.0709x over previous
import functools
import math

import jax
import jax.numpy as jnp
import numpy as np
from jax import lax
from jax.experimental import pallas as pl
from jax.experimental.pallas import tpu as pltpu
from jax.experimental.pallas import tpu_sc as plsc

VOCAB = 100000
D_MODEL = 768
MAX_POS = 2048
_SCALE = math.sqrt(float(D_MODEL))
_LANES = 16
_CHUNK = 32


def _positional_encoding_np(length, depth):
    depth_h = depth / 2
    positions = np.arange(length)[:, np.newaxis]
    depths = np.arange(depth_h)[np.newaxis, :] / depth_h
    angle_rates = 1 / 10000 ** depths
    angle_rads = positions * angle_rates
    return np.concatenate(
        [np.sin(angle_rads), np.cos(angle_rads)], axis=-1
    ).astype(np.float32)


@functools.partial(jax.jit, static_argnums=(3, 4))
def _run(xf, pos, table, batch, seq_len):
    info = plsc.get_sparse_core_info()
    nc, ns = info.num_cores, info.num_subcores
    nw = nc * ns
    t_span = seq_len // nw
    b_per_w = batch * t_span
    halves = t_span // _CHUNK
    n_chunks = batch * halves
    cols16 = D_MODEL // _LANES
    n_rows = batch * seq_len

    mesh = plsc.VectorSubcoreMesh(core_axis_name="c", subcore_axis_name="s")

    @functools.partial(
        pl.kernel,
        mesh=mesh,
        out_type=jax.ShapeDtypeStruct((n_rows, D_MODEL), jnp.float32),
        scratch_types=[
            pltpu.VMEM((b_per_w,), jnp.int32),
            pltpu.VMEM((_CHUNK, D_MODEL), jnp.float32),
            pltpu.VMEM((_CHUNK, D_MODEL), jnp.float32),
            pltpu.VMEM((_CHUNK, D_MODEL), jnp.float32),
            pltpu.VMEM((_CHUNK, D_MODEL), jnp.float32),
            pltpu.VMEM((_CHUNK, D_MODEL), jnp.float32),
            pltpu.SemaphoreType.DMA,
            pltpu.SemaphoreType.DMA,
            pltpu.SemaphoreType.DMA,
            pltpu.SemaphoreType.DMA,
            pltpu.SemaphoreType.DMA,
        ],
    )
    def body(x_hbm, pos_hbm, table_hbm, out_hbm,
             idx_v, pos_v, g0, g1, p0, p1,
             psem, gs0, gs1, os0, os1):
        g = (g0, g1)
        p = (p0, p1)
        gsem = (gs0, gs1)
        osem = (os0, os1)
        wid = lax.axis_index("s") * nc + lax.axis_index("c")
        t0 = wid * t_span
        for b in range(batch):
            pltpu.sync_copy(
                x_hbm.at[pl.ds(b * seq_len + t0, t_span)],
                idx_v.at[pl.ds(b * t_span, t_span)])

        def seg(j):
            return j % batch, j // batch

        def start_gather(j):
            b, half = seg(j)
            return pltpu.async_copy(
                table_hbm.at[
                    idx_v.at[pl.ds(b * t_span + half * _CHUNK, _CHUNK)]],
                g[j % 2], gsem[j % 2])

        ph = pltpu.async_copy(pos_hbm.at[pl.ds(t0, _CHUNK)], pos_v, psem)
        store_h = [None, None]
        pend = start_gather(0)
        for j in range(n_chunks):
            buf = j % 2
            b, half = seg(j)
            if j + 1 < n_chunks:
                nbuf = (j + 1) % 2
                if store_h[nbuf] is not None:
                    store_h[nbuf].wait()
                    store_h[nbuf] = None
                nxt = start_gather(j + 1)
            pend.wait()
            if j % batch == 0 and ph is not None:
                ph.wait()
                ph = None

            @plsc.parallel_loop(0, _CHUNK, 1, unroll=1)
            def _(r):
                for c in range(cols16):
                    sl = pl.ds(c * _LANES, _LANES)
                    p[buf][r, sl] = g[buf][r, sl] * _SCALE + pos_v[r, sl]

            if j == batch - 1 and halves > 1:
                ph = pltpu.async_copy(
                    pos_hbm.at[pl.ds(t0 + _CHUNK, _CHUNK)], pos_v, psem)
            store_h[buf] = pltpu.async_copy(
                p[buf],
                out_hbm.at[pl.ds(b * seq_len + t0 + half * _CHUNK, _CHUNK)],
                osem[buf])
            if j + 1 < n_chunks:
                pend = nxt
        for h in store_h:
            if h is not None:
                h.wait()

    return body(xf, pos, table)


def kernel(x, table):
    b, t = x.shape
    xf = x.reshape(b * t).astype(jnp.int32)
    pos = jnp.asarray(_positional_encoding_np(MAX_POS, D_MODEL))
    out = _run(xf, pos, table, b, t)
    return out.reshape(b, t, D_MODEL)


# speedup vs baseline: 1.1253x; 1.0509x over previous
import functools
import math

import jax
import jax.numpy as jnp
import numpy as np
from jax import lax
from jax.experimental import pallas as pl
from jax.experimental.pallas import tpu as pltpu
from jax.experimental.pallas import tpu_sc as plsc

VOCAB = 100000
D_MODEL = 768
MAX_POS = 2048
_SCALE = math.sqrt(float(D_MODEL))
_LANES = 16
_CHUNK = 16
_DEPTH = 4


def _positional_encoding_np(length, depth):
    depth_h = depth / 2
    positions = np.arange(length)[:, np.newaxis]
    depths = np.arange(depth_h)[np.newaxis, :] / depth_h
    angle_rates = 1 / 10000 ** depths
    angle_rads = positions * angle_rates
    return np.concatenate(
        [np.sin(angle_rads), np.cos(angle_rads)], axis=-1
    ).astype(np.float32)


@functools.partial(jax.jit, static_argnums=(3, 4))
def _run(xr, pos, table, batch, seq_len):
    info = plsc.get_sparse_core_info()
    nc, ns = info.num_cores, info.num_subcores
    nw = nc * ns
    t_span = seq_len // nw
    b_per_w = batch * t_span
    quarters = t_span // _CHUNK
    n_chunks = batch * quarters
    cols16 = D_MODEL // _LANES
    n_rows = batch * seq_len

    mesh = plsc.VectorSubcoreMesh(core_axis_name="c", subcore_axis_name="s")

    @functools.partial(
        pl.kernel,
        mesh=mesh,
        out_type=jax.ShapeDtypeStruct((n_rows, D_MODEL), jnp.float32),
        scratch_types=[
            pltpu.VMEM((b_per_w,), jnp.int32),
            pltpu.VMEM((_CHUNK, D_MODEL), jnp.float32),
            pltpu.VMEM((_CHUNK, D_MODEL), jnp.float32),
        ]
        + [pltpu.VMEM((_CHUNK, D_MODEL), jnp.float32)] * (2 * _DEPTH)
        + [pltpu.SemaphoreType.DMA] * (2 + 2 * _DEPTH),
    )
    def body(x_hbm, pos_hbm, table_hbm, out_hbm,
             idx_v, pb0, pb1,
             g0, g1, g2, g3, p0, p1, p2, p3,
             pbs0, pbs1, gs0, gs1, gs2, gs3, os0, os1, os2, os3):
        posb = (pb0, pb1)
        g = (g0, g1, g2, g3)
        p = (p0, p1, p2, p3)
        psem = (pbs0, pbs1)
        gsem = (gs0, gs1, gs2, gs3)
        osem = (os0, os1, os2, os3)
        wid = lax.axis_index("s") * nc + lax.axis_index("c")
        t0 = wid * t_span
        pltpu.sync_copy(x_hbm.at[pl.ds(wid * b_per_w, b_per_w)], idx_v)

        def seg(j):
            return j % batch, j // batch

        def start_gather(j):
            b, q = seg(j)
            return pltpu.async_copy(
                table_hbm.at[
                    idx_v.at[pl.ds(b * t_span + q * _CHUNK, _CHUNK)]],
                g[j % _DEPTH], gsem[j % _DEPTH])

        def start_pos(q):
            return pltpu.async_copy(
                pos_hbm.at[pl.ds(t0 + q * _CHUNK, _CHUNK)],
                posb[q % 2], psem[q % 2])

        ph = [start_pos(0), None]
        gh = [None] * _DEPTH
        sh = [None] * _DEPTH
        for j in range(min(_DEPTH - 1, n_chunks)):
            gh[j % _DEPTH] = start_gather(j)
        for j in range(n_chunks):
            buf = j % _DEPTH
            b, q = seg(j)
            if j + _DEPTH - 1 < n_chunks:
                nb = (j + _DEPTH - 1) % _DEPTH
                gh[nb] = start_gather(j + _DEPTH - 1)
            gh[buf].wait()
            if j % batch == 0:
                ph[q % 2].wait()
                if q + 1 < quarters:
                    ph[(q + 1) % 2] = start_pos(q + 1)
            if sh[buf] is not None:
                sh[buf].wait()
                sh[buf] = None

            @plsc.parallel_loop(0, _CHUNK, 1, unroll=1)
            def _(r):
                for c in range(cols16):
                    sl = pl.ds(c * _LANES, _LANES)
                    p[buf][r, sl] = (
                        g[buf][r, sl] * _SCALE + posb[q % 2][r, sl])

            sh[buf] = pltpu.async_copy(
                p[buf],
                out_hbm.at[pl.ds(b * seq_len + t0 + q * _CHUNK, _CHUNK)],
                osem[buf])
        for h in sh:
            if h is not None:
                h.wait()

    return body(xr, pos, table)


def kernel(x, table):
    b, t = x.shape
    nw = 32
    t_span = t // nw
    xr = (x.reshape(b, nw, t_span).transpose(1, 0, 2).reshape(b * t)
          .astype(jnp.int32))
    pos = jnp.asarray(_positional_encoding_np(MAX_POS, D_MODEL))
    out = _run(xr, pos, table, b, t)
    return out.reshape(b, t, D_MODEL)
